# Initial kernel scaffold; baseline (speedup 1.0000x reference)
import functools

import jax
import jax.numpy as jnp
from jax import lax
from jax.experimental import pallas as pl
from jax.experimental.pallas import tpu as pltpu
from jax.experimental.pallas import tpu_sc as plsc

HID = 128
SEQ = 200
EPS = 1e-12
NUM_CORES = 2
NUM_SUBCORES = 16
NW = NUM_CORES * NUM_SUBCORES
LANES = 16
VPR = HID // LANES
CHUNK = 128


def _rsqrt(x):
    i = lax.bitcast_convert_type(x, jnp.int32)
    i = jnp.int32(0x5F3759DF) - lax.shift_right_logical(i, 1)
    y = lax.bitcast_convert_type(i, jnp.float32)
    half = x * 0.5
    for _ in range(3):
        y = y * (1.5 - half * y * y)
    return y


def _body(tok_hbm, seg_hbm, word_hbm, pt_hbm, gam_hbm, bet_hbm, out_hbm,
          tokbuf, eidxbuf, wbuf, ebuf, gbuf, bbuf, sem_w, sem_e):
    n_total = out_hbm.shape[0]
    rows_per_w = n_total // NW
    n_chunks = rows_per_w // CHUNK

    wid = lax.axis_index("s") * NUM_CORES + lax.axis_index("c")
    base = wid * rows_per_w

    pltpu.sync_copy(gam_hbm, gbuf)
    pltpu.sync_copy(bet_hbm, bbuf)
    gam = [gbuf[pl.ds(LANES * j, LANES)] for j in range(VPR)]
    bet = [bbuf[pl.ds(LANES * j, LANES)] for j in range(VPR)]
    iota = lax.iota(jnp.int32, LANES)

    def chunk_body(c, carry):
        cb = base + c * CHUNK
        pltpu.sync_copy(tok_hbm.at[pl.ds(cb, CHUNK)], tokbuf)
        pltpu.sync_copy(seg_hbm.at[pl.ds(cb, CHUNK)], eidxbuf)
        for i in range(CHUNK // LANES):
            seg_v = eidxbuf[pl.ds(i * LANES, LANES)]
            l_v = lax.rem(cb + i * LANES + iota, jnp.int32(SEQ))
            eidxbuf[pl.ds(i * LANES, LANES)] = seg_v * SEQ + l_v
        cp_w = pltpu.async_copy(word_hbm.at[tokbuf], wbuf, sem_w)
        cp_e = pltpu.async_copy(pt_hbm.at[eidxbuf], ebuf, sem_e)
        cp_w.wait()
        cp_e.wait()

        def row_body(r, carry2):
            x = [wbuf[r, pl.ds(LANES * j, LANES)]
                 + ebuf[r, pl.ds(LANES * j, LANES)] for j in range(VPR)]
            s = x[0]
            sq = x[0] * x[0]
            for j in range(1, VPR):
                s = s + x[j]
                sq = sq + x[j] * x[j]
            mu = jnp.sum(s) * (1.0 / HID)
            ms = jnp.sum(sq) * (1.0 / HID)
            rstd = _rsqrt(ms - mu * mu + EPS)
            mu_v = lax.broadcast(mu, (LANES,))
            rstd_v = lax.broadcast(rstd, (LANES,))
            for j in range(VPR):
                wbuf[r, pl.ds(LANES * j, LANES)] = (
                    (x[j] - mu_v) * rstd_v * gam[j] + bet[j])
            return carry2

        lax.fori_loop(0, CHUNK, row_body, 0)
        pltpu.sync_copy(wbuf, out_hbm.at[pl.ds(cb, CHUNK)])
        return carry

    lax.fori_loop(0, n_chunks, chunk_body, 0)


def _make_kernel(n_total, interpret=False):
    mesh = plsc.VectorSubcoreMesh(core_axis_name="c", subcore_axis_name="s")
    return functools.partial(
        pl.kernel,
        out_type=jax.ShapeDtypeStruct((n_total, HID), jnp.float32),
        mesh=mesh,
        scratch_types=[
            pltpu.VMEM((CHUNK,), jnp.int32),
            pltpu.VMEM((CHUNK,), jnp.int32),
            pltpu.VMEM((CHUNK, HID), jnp.float32),
            pltpu.VMEM((CHUNK, HID), jnp.float32),
            pltpu.VMEM((HID,), jnp.float32),
            pltpu.VMEM((HID,), jnp.float32),
            pltpu.SemaphoreType.DMA,
            pltpu.SemaphoreType.DMA,
        ],
        interpret=interpret,
    )(_body)


@jax.jit
def kernel(tokens, segments, word_emb, pos_emb, type_emb, gamma, beta):
    b, l = tokens.shape
    n_total = b * l
    tok = tokens.reshape(n_total).astype(jnp.int32)
    seg = segments.reshape(n_total).astype(jnp.int32)
    pt = (type_emb[:, None, :] + pos_emb[None, :l, :]).reshape(2 * l, HID)
    out = _make_kernel(n_total)(
        tok, seg, word_emb, pt,
        gamma.astype(jnp.float32), beta.astype(jnp.float32))
    return out.reshape(b, l, HID)


# speedup vs baseline: 3.4877x; 3.4877x over previous
import functools

import jax
import jax.numpy as jnp
from jax import lax
from jax.experimental import pallas as pl
from jax.experimental.pallas import tpu as pltpu
from jax.experimental.pallas import tpu_sc as plsc

HID = 128
SEQ = 200
EPS = 1e-12
NUM_CORES = 2
NUM_SUBCORES = 16
NW = NUM_CORES * NUM_SUBCORES
LANES = 16
VPR = HID // LANES
CHUNK = 128


def _lane_sum(v, perms):
    for p in perms:
        v = v + v.at[p].get(mode=lax.GatherScatterMode.PROMISE_IN_BOUNDS)
    return v


def _rsqrt(x):
    i = lax.bitcast_convert_type(x, jnp.int32)
    i = jnp.int32(0x5F3759DF) - lax.shift_right_logical(i, 1)
    y = lax.bitcast_convert_type(i, jnp.float32)
    half = x * 0.5
    for _ in range(3):
        y = y * (1.5 - half * y * y)
    return y


def _worker_id():
    return lax.axis_index("s") * NUM_CORES + lax.axis_index("c")


def _body(tok_hbm, seg_hbm, word_hbm, pt_hbm, gam_hbm, bet_hbm, out_hbm,
          tokbuf, eidxbuf, wbuf, ebuf, gbuf, bbuf, sem_w, sem_e):
    n_total = out_hbm.shape[0]
    rows_per_w = n_total // NW
    n_chunks = rows_per_w // CHUNK

    wid = _worker_id()
    base = wid * rows_per_w

    pltpu.sync_copy(gam_hbm, gbuf)
    pltpu.sync_copy(bet_hbm, bbuf)
    gam = [gbuf[pl.ds(LANES * j, LANES)] for j in range(VPR)]
    bet = [bbuf[pl.ds(LANES * j, LANES)] for j in range(VPR)]
    iota = lax.iota(jnp.int32, LANES)
    perms = [lax.bitwise_xor(iota, jnp.int32(k)) for k in (8, 4, 2, 1)]

    def chunk_body(c, carry):
        cb = base + c * CHUNK
        pltpu.sync_copy(tok_hbm.at[pl.ds(cb, CHUNK)], tokbuf)
        pltpu.sync_copy(seg_hbm.at[pl.ds(cb, CHUNK)], eidxbuf)
        for i in range(CHUNK // LANES):
            seg_v = eidxbuf[pl.ds(i * LANES, LANES)]
            l_v = lax.rem(cb + i * LANES + iota, jnp.int32(SEQ))
            eidxbuf[pl.ds(i * LANES, LANES)] = seg_v * SEQ + l_v
        cp_w = pltpu.async_copy(word_hbm.at[tokbuf], wbuf, sem_w)
        cp_e = pltpu.async_copy(pt_hbm.at[eidxbuf], ebuf, sem_e)
        cp_w.wait()
        cp_e.wait()

        def row_body(r, carry2):
            x = [wbuf[r, pl.ds(LANES * j, LANES)]
                 + ebuf[r, pl.ds(LANES * j, LANES)] for j in range(VPR)]
            s = x[0]
            sq = x[0] * x[0]
            for j in range(1, VPR):
                s = s + x[j]
                sq = sq + x[j] * x[j]
            mu_v = _lane_sum(s, perms) * (1.0 / HID)
            ms_v = _lane_sum(sq, perms) * (1.0 / HID)
            rstd_v = _rsqrt(ms_v - mu_v * mu_v + EPS)
            for j in range(VPR):
                wbuf[r, pl.ds(LANES * j, LANES)] = (
                    (x[j] - mu_v) * rstd_v * gam[j] + bet[j])
            return carry2

        lax.fori_loop(0, CHUNK, row_body, 0)
        pltpu.sync_copy(wbuf, out_hbm.at[pl.ds(cb, CHUNK)])
        return carry

    lax.fori_loop(0, n_chunks, chunk_body, 0)


def _make_kernel(n_total, interpret=False):
    mesh = plsc.VectorSubcoreMesh(
        core_axis_name="c", subcore_axis_name="s",
        num_cores=NUM_CORES, num_subcores=NUM_SUBCORES)
    return functools.partial(
        pl.kernel,
        out_type=jax.ShapeDtypeStruct((n_total, HID), jnp.float32),
        mesh=mesh,
        scratch_types=[
            pltpu.VMEM((CHUNK,), jnp.int32),
            pltpu.VMEM((CHUNK,), jnp.int32),
            pltpu.VMEM((CHUNK, HID), jnp.float32),
            pltpu.VMEM((CHUNK, HID), jnp.float32),
            pltpu.VMEM((HID,), jnp.float32),
            pltpu.VMEM((HID,), jnp.float32),
            pltpu.SemaphoreType.DMA,
            pltpu.SemaphoreType.DMA,
        ],
        interpret=interpret,
    )(_body)


@jax.jit
def kernel(tokens, segments, word_emb, pos_emb, type_emb, gamma, beta):
    b, l = tokens.shape
    n_total = b * l
    tok = tokens.reshape(n_total).astype(jnp.int32)
    seg = segments.reshape(n_total).astype(jnp.int32)
    pt = (type_emb[:, None, :] + pos_emb[None, :l, :]).reshape(2 * l, HID)
    out = _make_kernel(n_total)(
        tok, seg, word_emb, pt,
        gamma.astype(jnp.float32), beta.astype(jnp.float32))
    return out.reshape(b, l, HID)


# speedup vs baseline: 8.3962x; 2.4074x over previous
import functools

import jax
import jax.numpy as jnp
from jax import lax
from jax.experimental import pallas as pl
from jax.experimental.pallas import tpu as pltpu
from jax.experimental.pallas import tpu_sc as plsc

HID = 128
SEQ = 200
EPS = 1e-12
NUM_CORES = 2
NUM_SUBCORES = 16
NW = NUM_CORES * NUM_SUBCORES
LANES = 16
VPR = HID // LANES
CHUNK = 128
UNROLL = 4


def _lane_sum(v, perms):
    for p in perms:
        v = v + v.at[p].get(mode=lax.GatherScatterMode.PROMISE_IN_BOUNDS)
    return v


def _rsqrt(x):
    i = lax.bitcast_convert_type(x, jnp.int32)
    i = jnp.int32(0x5F3759DF) - lax.shift_right_logical(i, 1)
    y = lax.bitcast_convert_type(i, jnp.float32)
    half = x * 0.5
    for _ in range(3):
        y = y * (1.5 - half * y * y)
    return y


def _worker_id():
    return lax.axis_index("s") * NUM_CORES + lax.axis_index("c")


def _body(tok_hbm, seg_hbm, word_hbm, pt_hbm, gam_hbm, bet_hbm, out_hbm,
          tok0, tok1, eidx0, eidx1, w0, w1, e0, e1, gbuf, bbuf,
          sem_t0, sem_t1, sem_s0, sem_s1, sem_w0, sem_w1,
          sem_e0, sem_e1, sem_o0, sem_o1):
    toks = (tok0, tok1)
    eidxs = (eidx0, eidx1)
    ws = (w0, w1)
    es = (e0, e1)
    sem_t = (sem_t0, sem_t1)
    sem_s = (sem_s0, sem_s1)
    sem_w = (sem_w0, sem_w1)
    sem_e = (sem_e0, sem_e1)
    sem_o = (sem_o0, sem_o1)

    n_total = out_hbm.shape[0]
    rows_per_w = n_total // NW
    n_chunks = rows_per_w // CHUNK

    wid = _worker_id()
    base = wid * rows_per_w

    pltpu.sync_copy(gam_hbm, gbuf)
    pltpu.sync_copy(bet_hbm, bbuf)
    gam = [gbuf[pl.ds(LANES * j, LANES)] for j in range(VPR)]
    bet = [bbuf[pl.ds(LANES * j, LANES)] for j in range(VPR)]
    iota = lax.iota(jnp.int32, LANES)
    perms = [lax.bitwise_xor(iota, jnp.int32(k)) for k in (8, 4, 2, 1)]

    def issue_idx(c, s):
        cb = base + c * CHUNK
        pltpu.async_copy(tok_hbm.at[pl.ds(cb, CHUNK)], toks[s], sem_t[s])
        pltpu.async_copy(seg_hbm.at[pl.ds(cb, CHUNK)], eidxs[s], sem_s[s])

    def wait_idx(s):
        pltpu.make_async_copy(
            tok_hbm.at[pl.ds(0, CHUNK)], toks[s], sem_t[s]).wait()
        pltpu.make_async_copy(
            seg_hbm.at[pl.ds(0, CHUNK)], eidxs[s], sem_s[s]).wait()

    def compute_eidx(c, s):
        cb = base + c * CHUNK
        for i in range(CHUNK // LANES):
            seg_v = eidxs[s][pl.ds(i * LANES, LANES)]
            l_v = lax.rem(cb + i * LANES + iota, jnp.int32(SEQ))
            eidxs[s][pl.ds(i * LANES, LANES)] = seg_v * SEQ + l_v

    def issue_gathers(s):
        pltpu.async_copy(word_hbm.at[toks[s]], ws[s], sem_w[s])
        pltpu.async_copy(pt_hbm.at[eidxs[s]], es[s], sem_e[s])

    def wait_gathers(s):
        pltpu.make_async_copy(
            word_hbm.at[pl.ds(0, CHUNK)], ws[s], sem_w[s]).wait()
        pltpu.make_async_copy(
            word_hbm.at[pl.ds(0, CHUNK)], es[s], sem_e[s]).wait()

    def issue_out(c, s):
        cb = base + c * CHUNK
        pltpu.async_copy(ws[s], out_hbm.at[pl.ds(cb, CHUNK)], sem_o[s])

    def wait_out(s):
        pltpu.make_async_copy(
            ws[s], out_hbm.at[pl.ds(0, CHUNK)], sem_o[s]).wait()

    def layernorm_chunk(s):
        wb, eb = ws[s], es[s]

        @plsc.parallel_loop(0, CHUNK, step=1, unroll=UNROLL)
        def row_body(r):
            x = [wb[r, pl.ds(LANES * j, LANES)]
                 + eb[r, pl.ds(LANES * j, LANES)] for j in range(VPR)]
            ssum = x[0]
            sq = x[0] * x[0]
            for j in range(1, VPR):
                ssum = ssum + x[j]
                sq = sq + x[j] * x[j]
            mu_v = _lane_sum(ssum, perms) * (1.0 / HID)
            ms_v = _lane_sum(sq, perms) * (1.0 / HID)
            rstd_v = _rsqrt(ms_v - mu_v * mu_v + EPS)
            for j in range(VPR):
                wb[r, pl.ds(LANES * j, LANES)] = (
                    (x[j] - mu_v) * rstd_v * gam[j] + bet[j])

    issue_idx(0, 0)
    issue_idx(1, 1)
    wait_idx(0)
    compute_eidx(0, 0)
    issue_gathers(0)

    def step(c, s):
        @pl.when(c + 1 <= n_chunks - 1)
        def _():
            o = 1 - s
            wait_idx(o)
            compute_eidx(c + 1, o)

            @pl.when(c >= 1)
            def _():
                wait_out(o)
            issue_gathers(o)

        wait_gathers(s)

        @pl.when(c + 2 <= n_chunks - 1)
        def _():
            issue_idx(c + 2, s)

        layernorm_chunk(s)
        issue_out(c, s)

    def pair_body(g, carry):
        step(2 * g, 0)
        step(2 * g + 1, 1)
        return carry

    lax.fori_loop(0, n_chunks // 2, pair_body, 0)
    wait_out(0)
    wait_out(1)


def _make_kernel(n_total, interpret=False):
    mesh = plsc.VectorSubcoreMesh(
        core_axis_name="c", subcore_axis_name="s",
        num_cores=NUM_CORES, num_subcores=NUM_SUBCORES)
    return functools.partial(
        pl.kernel,
        out_type=jax.ShapeDtypeStruct((n_total, HID), jnp.float32),
        mesh=mesh,
        scratch_types=[
            pltpu.VMEM((CHUNK,), jnp.int32),
            pltpu.VMEM((CHUNK,), jnp.int32),
            pltpu.VMEM((CHUNK,), jnp.int32),
            pltpu.VMEM((CHUNK,), jnp.int32),
            pltpu.VMEM((CHUNK, HID), jnp.float32),
            pltpu.VMEM((CHUNK, HID), jnp.float32),
            pltpu.VMEM((CHUNK, HID), jnp.float32),
            pltpu.VMEM((CHUNK, HID), jnp.float32),
            pltpu.VMEM((HID,), jnp.float32),
            pltpu.VMEM((HID,), jnp.float32),
        ] + [pltpu.SemaphoreType.DMA] * 10,
        interpret=interpret,
    )(_body)


@jax.jit
def kernel(tokens, segments, word_emb, pos_emb, type_emb, gamma, beta):
    b, l = tokens.shape
    n_total = b * l
    tok = tokens.reshape(n_total).astype(jnp.int32)
    seg = segments.reshape(n_total).astype(jnp.int32)
    pt = (type_emb[:, None, :] + pos_emb[None, :l, :]).reshape(2 * l, HID)
    out = _make_kernel(n_total)(
        tok, seg, word_emb, pt,
        gamma.astype(jnp.float32), beta.astype(jnp.float32))
    return out.reshape(b, l, HID)
